# Initial kernel scaffold; baseline (speedup 1.0000x reference)
import functools

import jax
import jax.numpy as jnp
from jax import lax
from jax.experimental import pallas as pl
from jax.experimental.pallas import tpu as pltpu
from jax.experimental.pallas import tpu_sc as plsc

B = 4096
KBLK = 8
MARGIN = jnp.float32(0.3)
NC = 2
NS = 16
L = 16
NW = NC * NS
ROWS_W = B // NW
CH_R = 8
N_CH = ROWS_W // CH_R
NV = B // L
INF = jnp.float32(jnp.inf)


def _tec_body(dm_hbm, out_hbm, buf, accv, sem):
    wid = lax.axis_index("s") * NC + lax.axis_index("c")
    row0 = wid * ROWS_W
    lane = lax.iota(jnp.int32, L)

    def chunk_body(ch, acc):
        base = row0 + ch * CH_R
        pltpu.sync_copy(dm_hbm.at[pl.ds(base, CH_R)], buf)

        def row_body(r, acc):
            i = base + r
            w0 = (i // L) * L
            v = buf[r, pl.ds(w0, L)]
            col = w0 + lane
            mask = ((col // KBLK) == (i // KBLK)) & (col != i)
            pos = jnp.max(jnp.where(mask, v, jnp.float32(0.0)))
            buf[r, pl.ds(w0, L)] = jnp.where(mask, INF, v)

            def min_body(c, carry):
                m1, m2 = carry
                x = buf[r, pl.ds(c * L, L)]
                return (jnp.minimum(m1, x),
                        jnp.minimum(m2, jnp.maximum(m1, x)))

            m1, m2 = lax.fori_loop(
                0, NV, min_body,
                (jnp.full((L,), INF), jnp.full((L,), INF)))

            g1 = jnp.min(m1)
            g1v = jnp.full((L,), g1)
            eq = m1 == g1v
            cnt = plsc.all_reduce_population_count(eq)
            second = jnp.min(jnp.where(eq, INF, m1))
            c2 = jnp.min(jnp.where(eq, m2, INF))
            negv = jnp.where(cnt >= 2, g1v,
                             jnp.full((L,), jnp.minimum(second, c2)))
            posv = jnp.full((L,), pos)
            loss = jnp.maximum(posv - negv + MARGIN, jnp.float32(0.0))
            return acc + loss

        return lax.fori_loop(0, CH_R, row_body, acc)

    acc = lax.fori_loop(0, N_CH, chunk_body, jnp.zeros((L,), jnp.float32))
    accv[...] = acc
    pltpu.sync_copy(accv, out_hbm.at[wid])


@jax.jit
def _sc_loss(distance_matrix):
    mesh = plsc.VectorSubcoreMesh(core_axis_name="c", subcore_axis_name="s")
    run = functools.partial(
        pl.kernel,
        mesh=mesh,
        out_type=jax.ShapeDtypeStruct((NW, L), jnp.float32),
        scratch_types=[
            pltpu.VMEM((CH_R, B), jnp.float32),
            pltpu.VMEM((L,), jnp.float32),
            pltpu.SemaphoreType.DMA,
        ],
    )(_tec_body)
    return run(distance_matrix)


def kernel(distance_matrix):
    partials = _sc_loss(distance_matrix)
    return jnp.sum(partials) / jnp.float32(B * L)


# speedup vs baseline: 27.0550x; 27.0550x over previous
import functools

import jax
import jax.numpy as jnp
from jax import lax
from jax.experimental import pallas as pl
from jax.experimental.pallas import tpu as pltpu
from jax.experimental.pallas import tpu_sc as plsc

B = 4096
KBLK = 8
MARGIN = 0.3
NC = 2
NS = 16
L = 16
NW = NC * NS
ROWS_W = B // NW
CH_R = 8
N_CH = ROWS_W // CH_R
NV = B // L
INF = float("inf")


def _tec_body(dm_hbm, out_hbm, buf, accv, sem):
    wid = lax.axis_index("s") * NC + lax.axis_index("c")
    row0 = wid * ROWS_W
    lane = lax.iota(jnp.int32, L)

    def chunk_body(ch, acc):
        base = row0 + ch * CH_R
        pltpu.sync_copy(dm_hbm.at[pl.ds(base, CH_R)], buf)

        def row_body(r, acc):
            i = base + r
            w0 = (i // L) * L
            v = buf[r, pl.ds(w0, L)]
            col = w0 + lane
            mask = ((col // KBLK) == (i // KBLK)) & (col != i)
            pos = jnp.max(jnp.where(mask, v, jnp.float32(0.0)))
            buf[r, pl.ds(w0, L)] = jnp.where(mask, INF, v)

            def min_body(c, carry):
                m1, m2 = carry
                x = buf[r, pl.ds(c * L, L)]
                return (jnp.minimum(m1, x),
                        jnp.minimum(m2, jnp.maximum(m1, x)))

            m1, m2 = lax.fori_loop(
                0, NV, min_body,
                (jnp.full((L,), INF), jnp.full((L,), INF)))

            g1 = jnp.min(m1)
            g1v = jnp.full((L,), g1)
            eq = m1 == g1v
            cnt = plsc.all_reduce_population_count(eq)
            second = jnp.min(jnp.where(eq, INF, m1))
            c2 = jnp.min(jnp.where(eq, m2, INF))
            negv = jnp.where(cnt >= 2, g1v,
                             jnp.full((L,), jnp.minimum(second, c2)))
            posv = jnp.full((L,), pos)
            loss = jnp.maximum(posv - negv + MARGIN, jnp.float32(0.0))
            return acc + loss

        return lax.fori_loop(0, CH_R, row_body, acc)

    acc = lax.fori_loop(0, N_CH, chunk_body, jnp.zeros((L,), jnp.float32))
    accv[...] = acc
    pltpu.sync_copy(accv, out_hbm.at[wid])


@jax.jit
def _sc_loss(distance_matrix):
    mesh = plsc.VectorSubcoreMesh(core_axis_name="c", subcore_axis_name="s")
    run = functools.partial(
        pl.kernel,
        mesh=mesh,
        out_type=jax.ShapeDtypeStruct((NW, L), jnp.float32),
        scratch_types=[
            pltpu.VMEM((CH_R, B), jnp.float32),
            pltpu.VMEM((L,), jnp.float32),
            pltpu.SemaphoreType.DMA,
        ],
        compiler_params=pltpu.CompilerParams(needs_layout_passes=False),
    )(_tec_body)
    return run(distance_matrix)


def kernel(distance_matrix):
    partials = _sc_loss(distance_matrix)
    return jnp.sum(partials) / jnp.float32(B * L)


# speedup vs baseline: 82.7359x; 3.0581x over previous
import functools

import jax
import jax.numpy as jnp
from jax import lax
from jax.experimental import pallas as pl
from jax.experimental.pallas import tpu as pltpu
from jax.experimental.pallas import tpu_sc as plsc

B = 4096
KBLK = 8
MARGIN = 0.3
NC = 2
NS = 16
L = 16
NW = NC * NS
ROWS_W = B // NW
CH_R = 8
N_CH = ROWS_W // CH_R
NV = B // L
U = 8
INF = float("inf")


def _merge2min(m1a, m2a, m1b, m2b):
    return (jnp.minimum(m1a, m1b),
            jnp.minimum(jnp.maximum(m1a, m1b), jnp.minimum(m2a, m2b)))


def _tec_body(dm_hbm, out_hbm, buf0, buf1, accv, sem0, sem1):
    wid = lax.axis_index("s") * NC + lax.axis_index("c")
    row0 = wid * ROWS_W
    lane = lax.iota(jnp.int32, L)
    bufs = (buf0, buf1)
    sems = (sem0, sem1)

    def make_row_body(buf, base):
        def row_body(r, acc):
            i = base + r
            w0 = (i // L) * L
            v = buf[r, pl.ds(w0, L)]
            col = w0 + lane
            mask = ((col // KBLK) == (i // KBLK)) & (col != i)
            pos = jnp.max(jnp.where(mask, v, jnp.float32(0.0)))
            buf[r, pl.ds(w0, L)] = jnp.where(mask, INF, v)

            def min_body(c, carry):
                ms = list(carry)
                off = c * (U * L)
                for u in range(U):
                    x = buf[r, pl.ds(off + u * L, L)]
                    k = u % 4
                    m1, m2 = ms[2 * k], ms[2 * k + 1]
                    ms[2 * k + 1] = jnp.minimum(m2, jnp.maximum(m1, x))
                    ms[2 * k] = jnp.minimum(m1, x)
                return tuple(ms)

            init = tuple(jnp.full((L,), INF) for _ in range(8))
            ms = lax.fori_loop(0, NV // U, min_body, init)
            m1a, m2a = _merge2min(*ms[0:4])
            m1b, m2b = _merge2min(*ms[4:8])
            m1, m2 = _merge2min(m1a, m2a, m1b, m2b)

            g1 = jnp.min(m1)
            g1v = jnp.full((L,), g1)
            eq = m1 == g1v
            cnt = plsc.all_reduce_population_count(eq)
            sc2 = jnp.min(jnp.minimum(jnp.where(eq, INF, m1),
                                      jnp.where(eq, m2, INF)))
            negv = jnp.where(cnt >= 2, g1v, jnp.full((L,), sc2))
            posv = jnp.full((L,), pos)
            loss = jnp.maximum(posv - negv + MARGIN, jnp.float32(0.0))
            return acc + loss
        return row_body

    acc = jnp.zeros((L,), jnp.float32)
    cp = pltpu.async_copy(dm_hbm.at[pl.ds(row0, CH_R)], buf0, sem0)
    for ch in range(N_CH):
        slot = ch % 2
        nxt = None
        if ch + 1 < N_CH:
            nslot = (ch + 1) % 2
            nxt = pltpu.async_copy(
                dm_hbm.at[pl.ds(row0 + (ch + 1) * CH_R, CH_R)],
                bufs[nslot], sems[nslot])
        cp.wait()
        acc = lax.fori_loop(
            0, CH_R, make_row_body(bufs[slot], row0 + ch * CH_R), acc)
        cp = nxt
    accv[...] = acc
    pltpu.sync_copy(accv, out_hbm.at[wid])


@jax.jit
def _sc_loss(distance_matrix):
    mesh = plsc.VectorSubcoreMesh(core_axis_name="c", subcore_axis_name="s")
    run = functools.partial(
        pl.kernel,
        mesh=mesh,
        out_type=jax.ShapeDtypeStruct((NW, L), jnp.float32),
        scratch_types=[
            pltpu.VMEM((CH_R, B), jnp.float32),
            pltpu.VMEM((CH_R, B), jnp.float32),
            pltpu.VMEM((L,), jnp.float32),
            pltpu.SemaphoreType.DMA,
            pltpu.SemaphoreType.DMA,
        ],
        compiler_params=pltpu.CompilerParams(needs_layout_passes=False),
    )(_tec_body)
    return run(distance_matrix)


def kernel(distance_matrix):
    partials = _sc_loss(distance_matrix)
    return jnp.sum(partials) / jnp.float32(B * L)
